# baseline (speedup 1.0000x reference)
import functools

import jax
import jax.numpy as jnp
from jax import lax
from jax.experimental import pallas as pl
from jax.experimental.pallas import tpu as pltpu
from jax.experimental.pallas import tpu_sc as plsc

L = 16
SCALE = 1.5
TAIL = 91 * 3
RSTART = 112 * 3


def _build(J, T, M):
    info = plsc.get_sparse_core_info()
    nc, ns = info.num_cores, info.num_subcores
    nw = nc * ns
    mesh = plsc.VectorSubcoreMesh(core_axis_name="c", subcore_axis_name="s")

    @functools.partial(
        pl.kernel,
        mesh=mesh,
        out_type=jax.ShapeDtypeStruct((J, T, M), jnp.float32),
        scratch_types=[
            pltpu.VMEM((T, 128), jnp.float32),
            pltpu.VMEM((T, 128), jnp.float32),
        ],
        compiler_params=pltpu.CompilerParams(use_tc_tiling_on_sc=True),
    )
    def run(x_hbm, o_hbm, buf, wbuf):
        wid = lax.axis_index("c") * ns + lax.axis_index("s")

        def plane_body(i, carry):
            j = i * nw + wid

            @pl.when(j < TAIL)
            def _():
                pltpu.sync_copy(x_hbm.at[j], o_hbm.at[j])

            @pl.when((j >= TAIL) & (j < J))
            def _():
                wj = jnp.where(j < RSTART,
                               TAIL + (j - TAIL) % 3,
                               RSTART + (j - RSTART) % 3)
                for m0 in (0, 128):
                    pltpu.sync_copy(x_hbm.at[j, :, pl.ds(m0, 128)], buf)
                    pltpu.sync_copy(x_hbm.at[wj, :, pl.ds(m0, 128)], wbuf)

                    def row_body(r, carry2):
                        for g in range(128 // L):
                            x = buf[r, pl.ds(g * L, L)]
                            w = wbuf[r, pl.ds(g * L, L)]
                            buf[r, pl.ds(g * L, L)] = (
                                x * SCALE - w * (SCALE - 1.0))
                        return carry2

                    lax.fori_loop(0, T, row_body, 0)
                    pltpu.sync_copy(buf, o_hbm.at[j, :, pl.ds(m0, 128)])
            return carry

        lax.fori_loop(0, (J + nw - 1) // nw, plane_body, 0)

    return run


def kernel(skeleton):
    M, T, V, C = skeleton.shape
    J = V * C
    xt = jnp.transpose(skeleton, (2, 3, 1, 0)).reshape(J, T, M)
    run = _build(J, T, M)
    out = run(xt)
    return out.reshape(V, C, T, M).transpose(3, 2, 0, 1)


# speedup vs baseline: 11.1647x; 11.1647x over previous
import functools

import jax
import jax.numpy as jnp
from jax import lax
from jax.experimental import pallas as pl
from jax.experimental.pallas import tpu as pltpu
from jax.experimental.pallas import tpu_sc as plsc

L = 16
SCALE = 1.5
TAIL = 91 * 3
RSTART = 112 * 3


def _build(J, T, M):
    info = plsc.get_sparse_core_info()
    nc, ns = info.num_cores, info.num_subcores
    nw = nc * ns
    mesh = plsc.VectorSubcoreMesh(core_axis_name="c", subcore_axis_name="s")

    @functools.partial(
        pl.kernel,
        mesh=mesh,
        out_type=jax.ShapeDtypeStruct((J, T, M), jnp.float32),
        scratch_types=[
            pltpu.VMEM((T, M), jnp.float32),
            pltpu.VMEM((T, 128), jnp.float32),
        ],
        compiler_params=pltpu.CompilerParams(use_tc_tiling_on_sc=True),
    )
    def run(x_hbm, o_hbm, buf, wbuf):
        wid = lax.axis_index("c") * ns + lax.axis_index("s")

        def plane_body(i, carry):
            j = i * nw + wid

            @pl.when(j < TAIL)
            def _():
                pltpu.sync_copy(x_hbm.at[j], buf)
                pltpu.sync_copy(buf, o_hbm.at[j])

            @pl.when((j >= TAIL) & (j < J))
            def _():
                wj = jnp.where(j < RSTART,
                               TAIL + (j - TAIL) % 3,
                               RSTART + (j - RSTART) % 3)
                for m0 in (0, 128):
                    pltpu.sync_copy(x_hbm.at[j, :, pl.ds(m0, 128)],
                                    buf.at[:, pl.ds(0, 128)])
                    pltpu.sync_copy(x_hbm.at[wj, :, pl.ds(m0, 128)], wbuf)

                    def row_body(r, carry2):
                        for g in range(128 // L):
                            x = buf[r, pl.ds(g * L, L)]
                            w = wbuf[r, pl.ds(g * L, L)]
                            buf[r, pl.ds(g * L, L)] = (
                                x * SCALE - w * (SCALE - 1.0))
                        return carry2

                    lax.fori_loop(0, T, row_body, 0)
                    pltpu.sync_copy(buf.at[:, pl.ds(0, 128)],
                                    o_hbm.at[j, :, pl.ds(m0, 128)])
            return carry

        lax.fori_loop(0, (J + nw - 1) // nw, plane_body, 0)

    return run


def kernel(skeleton):
    M, T, V, C = skeleton.shape
    J = V * C
    xt = jnp.transpose(skeleton, (2, 3, 1, 0)).reshape(J, T, M)
    run = _build(J, T, M)
    out = run(xt)
    return out.reshape(V, C, T, M).transpose(3, 2, 0, 1)


# speedup vs baseline: 30.5741x; 2.7385x over previous
import jax
import jax.numpy as jnp
from jax.experimental import pallas as pl

SCALE = 1.5
TAIL = 91 * 3
RSTART = 112 * 3
BLK = 7


def _body(x_ref, w_ref, o_ref):
    jb = pl.program_id(0)

    @pl.when(jb < TAIL // BLK)
    def _():
        o_ref[...] = x_ref[...]

    @pl.when(jb >= TAIL // BLK)
    def _():
        for p in range(BLK):
            j = jb * BLK + p
            widx = (j - jnp.where(j < RSTART, TAIL, RSTART)) % 3
            w = w_ref[pl.ds(widx, 1), :, :]
            o_ref[pl.ds(p, 1), :, :] = (
                x_ref[pl.ds(p, 1), :, :] * SCALE - w * (SCALE - 1.0))


def _wmap(j):
    return (jnp.where(j < RSTART // BLK, TAIL // BLK, RSTART // BLK), 0, 0)


def kernel(skeleton):
    M, T, V, C = skeleton.shape
    J = V * C
    xt = jnp.transpose(skeleton, (2, 3, 1, 0)).reshape(J, T, M)
    out = pl.pallas_call(
        _body,
        grid=(J // BLK,),
        in_specs=[
            pl.BlockSpec((BLK, T, M), lambda j: (j, 0, 0)),
            pl.BlockSpec((BLK, T, M), _wmap),
        ],
        out_specs=pl.BlockSpec((BLK, T, M), lambda j: (j, 0, 0)),
        out_shape=jax.ShapeDtypeStruct((J, T, M), jnp.float32),
    )(xt, xt)
    return out.reshape(V, C, T, M).transpose(3, 2, 0, 1)


# speedup vs baseline: 30.9635x; 1.0127x over previous
import jax
import jax.numpy as jnp
from jax.experimental import pallas as pl

SCALE = 1.5
TAIL = 91 * 3
RSTART = 112 * 3
BLK = 7


def _body(x_ref, w_ref, o_ref):
    jb = pl.program_id(0)

    @pl.when(jb < TAIL // BLK)
    def _():
        o_ref[...] = x_ref[...]

    @pl.when(jb >= TAIL // BLK)
    def _():
        for p in range(BLK):
            j = jb * BLK + p
            widx = (j - jnp.where(j < RSTART, TAIL, RSTART)) % 3
            w = w_ref[pl.ds(widx, 1), :, :]
            o_ref[pl.ds(p, 1), :, :] = (
                x_ref[pl.ds(p, 1), :, :] * SCALE - w * (SCALE - 1.0))


def _wmap(j):
    return (jnp.where(j < RSTART // BLK, TAIL // 3, RSTART // 3), 0, 0)


def kernel(skeleton):
    M, T, V, C = skeleton.shape
    J = V * C
    xt = jnp.transpose(skeleton, (2, 3, 1, 0)).reshape(J, T, M)
    out = pl.pallas_call(
        _body,
        grid=(J // BLK,),
        in_specs=[
            pl.BlockSpec((BLK, T, M), lambda j: (j, 0, 0)),
            pl.BlockSpec((3, T, M), _wmap),
        ],
        out_specs=pl.BlockSpec((BLK, T, M), lambda j: (j, 0, 0)),
        out_shape=jax.ShapeDtypeStruct((J, T, M), jnp.float32),
    )(xt, xt)
    return out.reshape(V, C, T, M).transpose(3, 2, 0, 1)


# speedup vs baseline: 34.0736x; 1.1004x over previous
import jax
import jax.numpy as jnp
from jax.experimental import pallas as pl
from jax.experimental.pallas import tpu as pltpu

SCALE = 1.5
TAIL = 91 * 3
RSTART = 112 * 3
BLK = 21


def _body(x_ref, w_ref, o_ref):
    jb = pl.program_id(0)

    @pl.when(jb < TAIL // BLK)
    def _():
        o_ref[...] = x_ref[...]

    @pl.when(jb >= TAIL // BLK)
    def _():
        for p in range(BLK):
            j = jb * BLK + p
            widx = (j - jnp.where(j < RSTART, TAIL, RSTART)) % 3
            w = w_ref[pl.ds(widx, 1), :, :]
            o_ref[pl.ds(p, 1), :, :] = (
                x_ref[pl.ds(p, 1), :, :] * SCALE - w * (SCALE - 1.0))


def _wmap(j):
    return (jnp.where(j < RSTART // BLK, TAIL // 3, RSTART // 3), 0, 0)


def kernel(skeleton):
    M, T, V, C = skeleton.shape
    J = V * C
    xt = jnp.transpose(skeleton, (2, 3, 1, 0)).reshape(J, T, M)
    out = pl.pallas_call(
        _body,
        grid=(J // BLK,),
        in_specs=[
            pl.BlockSpec((BLK, T, M), lambda j: (j, 0, 0)),
            pl.BlockSpec((3, T, M), _wmap),
        ],
        out_specs=pl.BlockSpec((BLK, T, M), lambda j: (j, 0, 0)),
        out_shape=jax.ShapeDtypeStruct((J, T, M), jnp.float32),
        compiler_params=pltpu.CompilerParams(
            vmem_limit_bytes=128 * 1024 * 1024),
    )(xt, xt)
    return out.reshape(V, C, T, M).transpose(3, 2, 0, 1)
